# Initial kernel scaffold; baseline (speedup 1.0000x reference)
import jax
import jax.numpy as jnp
from jax.experimental import pallas as pl


def kernel(input_ids, char_table, pos_table):
    raise NotImplementedError("write your pallas kernel here")


# speedup vs baseline: 1.0105x; 1.0105x over previous
import functools

import jax
import jax.numpy as jnp
from jax import lax
from jax.experimental import pallas as pl
from jax.experimental.pallas import tpu as pltpu
from jax.experimental.pallas import tpu_sc as plsc

_NC = 2
_NS = 16
_NW = _NC * _NS
_CHUNK = 128
_LANES = 16


def _emb_body(nchunk, seq_len, dim, ids_hbm, table_hbm, pos_hbm, out_hbm,
              idx_v, buf0, buf1, pos_v, sem0, sem1):
    c = lax.axis_index("c")
    s = lax.axis_index("s")
    wid = s * _NC + c
    rows_per_w = nchunk * _CHUNK
    base = wid * rows_per_w

    pltpu.sync_copy(ids_hbm.at[wid], idx_v)
    pltpu.sync_copy(pos_hbm, pos_v)

    bufs = (buf0, buf1)
    sems = (sem0, sem1)

    pltpu.async_copy(table_hbm.at[idx_v.at[0]], bufs[0], sems[0])

    for cidx in range(nchunk):
        buf = bufs[cidx % 2]
        sem = sems[cidx % 2]
        pltpu.make_async_copy(table_hbm.at[idx_v.at[cidx]], buf, sem).wait()
        if cidx + 1 < nchunk:
            pltpu.async_copy(
                table_hbm.at[idx_v.at[cidx + 1]], bufs[(cidx + 1) % 2],
                sems[(cidx + 1) % 2])
        pos_base = (cidx * _CHUNK) % seq_len

        def add_row(r, carry, buf=buf, pos_base=pos_base):
            for d in range(dim // _LANES):
                v = pos_v[pos_base + r, pl.ds(d * _LANES, _LANES)]
                plsc.addupdate(buf.at[r, pl.ds(d * _LANES, _LANES)], v)
            return carry

        lax.fori_loop(0, _CHUNK, add_row, 0)
        pltpu.sync_copy(buf, out_hbm.at[pl.ds(base + cidx * _CHUNK, _CHUNK)])


def kernel(input_ids, char_table, pos_table):
    bsz, seq_len = input_ids.shape
    vocab, dim = char_table.shape
    total = bsz * seq_len
    rows_per_w = total // _NW
    nchunk = rows_per_w // _CHUNK

    ids3 = input_ids.reshape(_NW, nchunk, _CHUNK)

    mesh = plsc.VectorSubcoreMesh(core_axis_name="c", subcore_axis_name="s")
    body = functools.partial(_emb_body, nchunk, seq_len, dim)
    out = pl.kernel(
        body,
        out_type=jax.ShapeDtypeStruct((total, dim), jnp.float32),
        mesh=mesh,
        scratch_types=[
            pltpu.VMEM((nchunk, _CHUNK), jnp.int32),
            pltpu.VMEM((_CHUNK, dim), jnp.float32),
            pltpu.VMEM((_CHUNK, dim), jnp.float32),
            pltpu.VMEM((seq_len, dim), jnp.float32),
            pltpu.SemaphoreType.DMA,
            pltpu.SemaphoreType.DMA,
        ],
    )(ids3, char_table, pos_table)
    return out.reshape(bsz, seq_len, dim)
